# baseline (speedup 1.0000x reference)
import functools

import jax
import jax.numpy as jnp
from jax.experimental import pallas as pl
from jax.experimental.pallas import tpu as pltpu

N = 2048
D = 1024
F = 2048
E = 8
R_DIM = 128
LANES = 128

TM_GATE = 256
TM = 256
TF = 512


def _gate_body(x_ref, wr_ref, br_ref, wg_ref, bg_ref, w_ref):
    r = jnp.tanh(jnp.dot(x_ref[...], wr_ref[...],
                         preferred_element_type=jnp.float32) + br_ref[...])
    logits = jnp.dot(r, wg_ref[...],
                     preferred_element_type=jnp.float32) + bg_ref[...]
    lane = jax.lax.broadcasted_iota(jnp.int32, logits.shape, 1)
    valid = lane < E
    logits = jnp.where(valid, logits, -1e30)
    m = jnp.max(logits, axis=1, keepdims=True)
    ex = jnp.where(valid, jnp.exp(logits - m), 0.0)
    ws = ex / jnp.sum(ex, axis=1, keepdims=True)
    m1 = jnp.max(ws, axis=1, keepdims=True)
    idx1 = jnp.min(jnp.where(ws == m1, lane, LANES), axis=1, keepdims=True)
    oh1 = lane == idx1
    ws_rest = jnp.where(oh1, -1.0, ws)
    m2 = jnp.max(ws_rest, axis=1, keepdims=True)
    idx2 = jnp.min(jnp.where(ws_rest == m2, lane, LANES), axis=1, keepdims=True)
    oh2 = lane == idx2
    wh = jnp.where(jnp.logical_or(oh1, oh2), ws, 0.0)
    wh = wh / (m1 + m2 + 1e-9)
    w_ref[...] = jnp.where(lane == E, 1.0, wh)


def _moe_body(w_ref, x_ref, w1_ref, b1_ref, w2_ref, b2_ref, o_ref):
    e = pl.program_id(1)
    f = pl.program_id(2)
    h = jax.nn.gelu(jnp.dot(x_ref[...], w1_ref[0],
                            preferred_element_type=jnp.float32) + b1_ref[0, 0, :])
    contrib = jnp.dot(h, w2_ref[0], preferred_element_type=jnp.float32)
    lane = jax.lax.broadcasted_iota(jnp.int32, w_ref.shape, 1)
    w_col = jnp.sum(jnp.where(lane == e, w_ref[...], 0.0), axis=1,
                    keepdims=True)
    acc = w_col * contrib

    @pl.when(jnp.logical_and(e == 0, f == 0))
    def _init():
        o_ref[...] = acc

    @pl.when(jnp.logical_not(jnp.logical_and(e == 0, f == 0)))
    def _acc():
        o_ref[...] += acc

    @pl.when(f == pl.num_programs(2) - 1)
    def _bias():
        o_ref[...] += w_col * b2_ref[0, 0, :]


@jax.jit
def kernel(x, Wr, br, Wg, bg, We1, be1, We2, be2, Ws1, bs1, Ws2, bs2):
    wg_p = jnp.zeros((R_DIM, LANES), jnp.float32).at[:, :E].set(Wg)
    bg_p = jnp.zeros((1, LANES), jnp.float32).at[:, :E].set(bg)
    w_hard = pl.pallas_call(
        _gate_body,
        grid=(N // TM_GATE,),
        in_specs=[
            pl.BlockSpec((TM_GATE, D), lambda m: (m, 0)),
            pl.BlockSpec((D, R_DIM), lambda m: (0, 0)),
            pl.BlockSpec((1, R_DIM), lambda m: (0, 0)),
            pl.BlockSpec((R_DIM, LANES), lambda m: (0, 0)),
            pl.BlockSpec((1, LANES), lambda m: (0, 0)),
        ],
        out_specs=pl.BlockSpec((TM_GATE, LANES), lambda m: (m, 0)),
        out_shape=jax.ShapeDtypeStruct((N, LANES), jnp.float32),
        compiler_params=pltpu.CompilerParams(
            dimension_semantics=("parallel",)),
    )(x, Wr, br.reshape(1, R_DIM), wg_p, bg_p)

    w1 = jnp.concatenate([We1, Ws1[None]], axis=0)
    b1 = jnp.concatenate([be1, bs1[None]], axis=0).reshape(E + 1, 1, F)
    w2 = jnp.concatenate([We2, Ws2[None]], axis=0)
    b2 = jnp.concatenate([be2, bs2[None]], axis=0).reshape(E + 1, 1, D)

    out = pl.pallas_call(
        _moe_body,
        grid=(N // TM, E + 1, F // TF),
        in_specs=[
            pl.BlockSpec((TM, LANES), lambda m, e, f: (m, 0)),
            pl.BlockSpec((TM, D), lambda m, e, f: (m, 0)),
            pl.BlockSpec((1, D, TF), lambda m, e, f: (e, 0, f)),
            pl.BlockSpec((1, 1, TF), lambda m, e, f: (e, 0, f)),
            pl.BlockSpec((1, TF, D), lambda m, e, f: (e, f, 0)),
            pl.BlockSpec((1, 1, D), lambda m, e, f: (e, 0, 0)),
        ],
        out_specs=pl.BlockSpec((TM, D), lambda m, e, f: (m, 0)),
        out_shape=jax.ShapeDtypeStruct((N, D), jnp.float32),
        compiler_params=pltpu.CompilerParams(
            dimension_semantics=("parallel", "arbitrary", "arbitrary")),
    )(w_hard, x, w1, b1, w2, b2)
    return out
